# Initial kernel scaffold; baseline (speedup 1.0000x reference)
import jax
import jax.numpy as jnp
from jax.experimental import pallas as pl


def kernel(features, edge_index, W1, b1, W2, b2):
    raise NotImplementedError("write your pallas kernel here")


# speedup vs baseline: 23.2468x; 23.2468x over previous
import functools

import jax
import jax.numpy as jnp
from jax import lax
from jax.experimental import pallas as pl
from jax.experimental.pallas import tpu as pltpu
from jax.experimental.pallas import tpu_sc as plsc

N = 10000
E = 320000
D_IN = 128
D_HID = 16
D_OUT = 40

NC = 2
NS = 16
NW = NC * NS
BATCH = 125
G = E // BATCH
MS = G // NW

N_PAD = 10240
RPT = N_PAD // NS
RING = 4
DEG_W = 4

assert E == NW * MS * BATCH
assert N_PAD % (NS * 8) == 0 and N_PAD >= N

_MESH = plsc.VectorSubcoreMesh(
    core_axis_name="c", subcore_axis_name="s", num_cores=NC, num_subcores=NS
)
_SC_PARAMS = pltpu.CompilerParams(use_tc_tiling_on_sc=False)


def _nrsqrt(x):
    xi = lax.bitcast_convert_type(x, jnp.int32)
    yi = jnp.int32(0x5F3759DF) - lax.shift_right_logical(xi, 1)
    y = lax.bitcast_convert_type(yi, jnp.float32)
    for _ in range(3):
        y = y * (1.5 - 0.5 * x * y * y)
    return y


def _mp_loop(table, sidx, didx, rows, acc_sh, gsem, ssem):
    for p in range(RING - 1):
        pltpu.async_copy(table.at[sidx.at[p]], rows.at[p], gsem)

    def body(j, carry):
        pltpu.make_async_copy(table.at[sidx.at[j]], rows.at[j % RING],
                              gsem).wait()

        @pl.when(j >= 1)
        def _():
            pltpu.make_async_copy(rows.at[(j - 1) % RING],
                                  acc_sh.at[didx.at[j - 1]], ssem).wait()

        @pl.when(j + RING - 1 < MS)
        def _():
            pltpu.async_copy(table.at[sidx.at[j + RING - 1]],
                             rows.at[(j + RING - 1) % RING], gsem)

        pltpu.async_copy(rows.at[j % RING], acc_sh.at[didx.at[j]], ssem,
                         add=True)
        return carry

    lax.fori_loop(0, MS, body, 0)
    pltpu.make_async_copy(rows.at[(MS - 1) % RING],
                          acc_sh.at[didx.at[MS - 1]], ssem).wait()


@functools.partial(
    pl.kernel,
    out_type=jax.ShapeDtypeStruct((NC, 2, N_PAD), jnp.float32),
    mesh=_MESH,
    compiler_params=_SC_PARAMS,
    scratch_types=[
        pltpu.VMEM((MS, BATCH), jnp.int32),
        pltpu.VMEM((MS, BATCH), jnp.int32),
        pltpu.VMEM((BATCH,), jnp.float32),
        pltpu.VMEM_SHARED((N_PAD,), jnp.float32),
        pltpu.VMEM_SHARED((N_PAD,), jnp.float32),
        pltpu.SemaphoreType.DMA,
        pltpu.SemaphoreType.DMA,
    ],
)
def _deg_kernel(src_hbm, dst_hbm, zeros_hbm, ones_hbm, out_hbm,
                sidx, didx, ones_v, dsrc_sh, ddst_sh, sema, semb):
    c = lax.axis_index("c")
    s = lax.axis_index("s")
    wid = c * NS + s
    z0 = s * RPT

    c1 = pltpu.async_copy(src_hbm.at[pl.ds(wid * MS, MS)], sidx, sema)
    c2 = pltpu.async_copy(dst_hbm.at[pl.ds(wid * MS, MS)], didx, sema)
    c3 = pltpu.async_copy(ones_hbm, ones_v, sema)
    c4 = pltpu.async_copy(zeros_hbm.at[pl.ds(z0, RPT)],
                          dsrc_sh.at[pl.ds(z0, RPT)], semb)
    c5 = pltpu.async_copy(zeros_hbm.at[pl.ds(z0, RPT)],
                          ddst_sh.at[pl.ds(z0, RPT)], semb)
    c1.wait(); c2.wait(); c3.wait(); c4.wait(); c5.wait()
    plsc.subcore_barrier()

    def body(j, carry):
        @pl.when(j >= DEG_W)
        def _():
            pltpu.make_async_copy(ones_v, dsrc_sh.at[sidx.at[j - DEG_W]],
                                  sema).wait()
            pltpu.make_async_copy(ones_v, ddst_sh.at[didx.at[j - DEG_W]],
                                  semb).wait()
        pltpu.async_copy(ones_v, dsrc_sh.at[sidx.at[j]], sema, add=True)
        pltpu.async_copy(ones_v, ddst_sh.at[didx.at[j]], semb, add=True)
        return carry

    lax.fori_loop(0, MS, body, 0)

    def drain(j, carry):
        pltpu.make_async_copy(ones_v, dsrc_sh.at[sidx.at[j]], sema).wait()
        pltpu.make_async_copy(ones_v, ddst_sh.at[didx.at[j]], semb).wait()
        return carry

    lax.fori_loop(MS - DEG_W, MS, drain, 0)
    plsc.subcore_barrier()

    w1 = pltpu.async_copy(dsrc_sh.at[pl.ds(z0, RPT)],
                          out_hbm.at[c, 0, pl.ds(z0, RPT)], sema)
    w2 = pltpu.async_copy(ddst_sh.at[pl.ds(z0, RPT)],
                          out_hbm.at[c, 1, pl.ds(z0, RPT)], semb)
    w1.wait(); w2.wait()


@functools.partial(
    pl.kernel,
    out_type=jax.ShapeDtypeStruct((NC, N_PAD, D_HID), jnp.float32),
    mesh=_MESH,
    compiler_params=_SC_PARAMS,
    scratch_types=[
        pltpu.VMEM((MS, BATCH), jnp.int32),
        pltpu.VMEM((MS, BATCH), jnp.int32),
        pltpu.VMEM((RING, BATCH, D_HID), jnp.float32),
        pltpu.VMEM_SHARED((N_PAD, D_HID), jnp.float32),
        pltpu.SemaphoreType.DMA,
        pltpu.SemaphoreType.DMA,
    ],
)
def _msgpass_kernel(h_hbm, src_hbm, dst_hbm, zeros_hbm, out_hbm,
                    sidx, didx, rows, acc_sh, gsem, ssem):
    c = lax.axis_index("c")
    s = lax.axis_index("s")
    wid = c * NS + s
    z0 = s * RPT

    c1 = pltpu.async_copy(src_hbm.at[pl.ds(wid * MS, MS)], sidx, gsem)
    c2 = pltpu.async_copy(dst_hbm.at[pl.ds(wid * MS, MS)], didx, gsem)
    c3 = pltpu.async_copy(zeros_hbm.at[pl.ds(z0, RPT)],
                          acc_sh.at[pl.ds(z0, RPT)], ssem)
    c1.wait(); c2.wait(); c3.wait()
    plsc.subcore_barrier()

    _mp_loop(h_hbm, sidx, didx, rows, acc_sh, gsem, ssem)
    plsc.subcore_barrier()

    pltpu.sync_copy(acc_sh.at[pl.ds(z0, RPT)], out_hbm.at[c, pl.ds(z0, RPT)])


@functools.partial(
    pl.kernel,
    out_type=(
        jax.ShapeDtypeStruct((NC, N_PAD, D_HID), jnp.float32),
        jax.ShapeDtypeStruct((NC, N_PAD, D_HID), jnp.float32),
    ),
    mesh=_MESH,
    compiler_params=_SC_PARAMS,
    scratch_types=[
        pltpu.VMEM((MS, BATCH), jnp.int32),
        pltpu.VMEM((MS, BATCH), jnp.int32),
        pltpu.VMEM((RPT, D_HID), jnp.float32),
        pltpu.VMEM((RPT, D_HID), jnp.float32),
        pltpu.VMEM((RPT,), jnp.float32),
        pltpu.VMEM((RPT,), jnp.float32),
        pltpu.VMEM((RPT,), jnp.float32),
        pltpu.VMEM((RPT,), jnp.float32),
        pltpu.VMEM((D_HID,), jnp.float32),
        pltpu.VMEM((RING, BATCH, D_HID), jnp.float32),
        pltpu.VMEM_SHARED((N_PAD, D_HID), jnp.float32),
        pltpu.SemaphoreType.DMA,
        pltpu.SemaphoreType.DMA,
    ],
)
def _sc_b(acc1_hbm, deg_hbm, b1_hbm, src_hbm, dst_hbm, zeros2_hbm,
          h2_hbm, acc_hbm,
          sidx, didx, a0buf, a1buf, o0buf, o1buf, i0buf, i1buf, b1buf, rows,
          acc_sh, gsem, ssem):
    c = lax.axis_index("c")
    s = lax.axis_index("s")
    wid = c * NS + s
    z0 = s * RPT

    cps = [
        pltpu.async_copy(src_hbm.at[pl.ds(wid * MS, MS)], sidx, gsem),
        pltpu.async_copy(dst_hbm.at[pl.ds(wid * MS, MS)], didx, gsem),
        pltpu.async_copy(acc1_hbm.at[0, pl.ds(z0, RPT)], a0buf, gsem),
        pltpu.async_copy(acc1_hbm.at[1, pl.ds(z0, RPT)], a1buf, gsem),
        pltpu.async_copy(deg_hbm.at[0, 0, pl.ds(z0, RPT)], o0buf, gsem),
        pltpu.async_copy(deg_hbm.at[1, 0, pl.ds(z0, RPT)], o1buf, gsem),
        pltpu.async_copy(deg_hbm.at[0, 1, pl.ds(z0, RPT)], i0buf, gsem),
        pltpu.async_copy(deg_hbm.at[1, 1, pl.ds(z0, RPT)], i1buf, gsem),
        pltpu.async_copy(b1_hbm, b1buf, gsem),
        pltpu.async_copy(zeros2_hbm.at[pl.ds(z0, RPT)],
                         acc_sh.at[pl.ds(z0, RPT)], ssem),
    ]
    for cp in cps:
        cp.wait()

    b1v = b1buf[...]

    def h2_group(g, carry):
        r0 = g * D_HID
        ndv = _nrsqrt(jnp.maximum(i0buf[pl.ds(r0, D_HID)]
                                  + i1buf[pl.ds(r0, D_HID)], 1.0))
        nsv = _nrsqrt(jnp.maximum(o0buf[pl.ds(r0, D_HID)]
                                  + o1buf[pl.ds(r0, D_HID)], 1.0))
        for k in range(D_HID):
            agg = a0buf[r0 + k, :] + a1buf[r0 + k, :]
            a0buf[r0 + k, :] = jnp.maximum(agg * ndv[k] + b1v, 0.0) * nsv[k]
        return carry

    lax.fori_loop(0, RPT // D_HID, h2_group, 0)
    pltpu.sync_copy(a0buf, h2_hbm.at[c, pl.ds(z0, RPT)])
    plsc.subcore_barrier()

    _mp_loop(h2_hbm.at[c], sidx, didx, rows, acc_sh, gsem, ssem)
    plsc.subcore_barrier()

    pltpu.sync_copy(acc_sh.at[pl.ds(z0, RPT)], acc_hbm.at[c, pl.ds(z0, RPT)])


def _d1_body(feat_ref, deg_ref, w1_ref, out_ref):
    odeg = deg_ref[0, 0] + deg_ref[1, 0]
    nsrc = lax.rsqrt(jnp.maximum(odeg[:N], 1.0))
    h = feat_ref[...] * nsrc[:, None]
    out_ref[:N] = jnp.dot(h, w1_ref[...], preferred_element_type=jnp.float32)
    out_ref[N:] = jnp.zeros((N_PAD - N, D_HID), jnp.float32)


def _d3_body(accp_ref, deg_ref, w2_ref, b2_ref, out_ref):
    agg = accp_ref[0, :N] + accp_ref[1, :N]
    ideg = deg_ref[0, 1, :N] + deg_ref[1, 1, :N]
    ndst = lax.rsqrt(jnp.maximum(ideg, 1.0))[:, None]
    mm = jnp.dot(agg, w2_ref[...], preferred_element_type=jnp.float32)
    out_ref[...] = mm * ndst + b2_ref[...]


_dense1 = pl.pallas_call(
    _d1_body, out_shape=jax.ShapeDtypeStruct((N_PAD, D_HID), jnp.float32))
_dense3 = pl.pallas_call(
    _d3_body, out_shape=jax.ShapeDtypeStruct((N, D_OUT), jnp.float32))


def kernel(features, edge_index, W1, b1, W2, b2):
    src = edge_index[0].reshape(G, BATCH)
    dst = edge_index[1].reshape(G, BATCH)
    zeros1 = jnp.zeros((N_PAD,), jnp.float32)
    zeros2 = jnp.zeros((N_PAD, D_HID), jnp.float32)
    ones_b = jnp.ones((BATCH,), jnp.float32)

    degp = _deg_kernel(src, dst, zeros1, ones_b)
    h1 = _dense1(features, degp, W1)
    acc1 = _msgpass_kernel(h1, src, dst, zeros2)
    _h2, acc2 = _sc_b(acc1, degp, b1, src, dst, zeros2)
    return _dense3(acc2, degp, W2, b2)


# speedup vs baseline: 29.5672x; 1.2719x over previous
import functools

import jax
import jax.numpy as jnp
from jax import lax
from jax.experimental import pallas as pl
from jax.experimental.pallas import tpu as pltpu
from jax.experimental.pallas import tpu_sc as plsc

N = 10000
E = 320000
D_IN = 128
D_HID = 16
D_OUT = 40

NC = 2
NS = 16
NW = NC * NS
BATCH = 500
G = E // BATCH
MS = G // NW

N_PAD = 10240
RPT = N_PAD // NS
RING = 4
DEG_W = 4

assert E == NW * MS * BATCH
assert N_PAD % (NS * 8) == 0 and N_PAD >= N

_MESH = plsc.VectorSubcoreMesh(
    core_axis_name="c", subcore_axis_name="s", num_cores=NC, num_subcores=NS
)
_SC_PARAMS = pltpu.CompilerParams(use_tc_tiling_on_sc=False)


def _nrsqrt(x):
    xi = lax.bitcast_convert_type(x, jnp.int32)
    yi = jnp.int32(0x5F3759DF) - lax.shift_right_logical(xi, 1)
    y = lax.bitcast_convert_type(yi, jnp.float32)
    for _ in range(3):
        y = y * (1.5 - 0.5 * x * y * y)
    return y


def _mp_loop(table, sidx, didx, rows, acc_sh, gsem, ssem):
    for p in range(RING - 1):
        pltpu.async_copy(table.at[sidx.at[p]], rows.at[p], gsem)

    def body(j, carry):
        pltpu.make_async_copy(table.at[sidx.at[j]], rows.at[j % RING],
                              gsem).wait()

        @pl.when(j >= 1)
        def _():
            pltpu.make_async_copy(rows.at[(j - 1) % RING],
                                  acc_sh.at[didx.at[j - 1]], ssem).wait()

        @pl.when(j + RING - 1 < MS)
        def _():
            pltpu.async_copy(table.at[sidx.at[j + RING - 1]],
                             rows.at[(j + RING - 1) % RING], gsem)

        pltpu.async_copy(rows.at[j % RING], acc_sh.at[didx.at[j]], ssem,
                         add=True)
        return carry

    lax.fori_loop(0, MS, body, 0)
    pltpu.make_async_copy(rows.at[(MS - 1) % RING],
                          acc_sh.at[didx.at[MS - 1]], ssem).wait()


@functools.partial(
    pl.kernel,
    out_type=jax.ShapeDtypeStruct((NC, 2, N_PAD), jnp.float32),
    mesh=_MESH,
    compiler_params=_SC_PARAMS,
    scratch_types=[
        pltpu.VMEM((MS, BATCH), jnp.int32),
        pltpu.VMEM((MS, BATCH), jnp.int32),
        pltpu.VMEM((BATCH,), jnp.float32),
        pltpu.VMEM_SHARED((N_PAD,), jnp.float32),
        pltpu.VMEM_SHARED((N_PAD,), jnp.float32),
        pltpu.SemaphoreType.DMA,
        pltpu.SemaphoreType.DMA,
    ],
)
def _deg_kernel(src_hbm, dst_hbm, zeros_hbm, ones_hbm, out_hbm,
                sidx, didx, ones_v, dsrc_sh, ddst_sh, sema, semb):
    c = lax.axis_index("c")
    s = lax.axis_index("s")
    wid = c * NS + s
    z0 = s * RPT

    c1 = pltpu.async_copy(src_hbm.at[pl.ds(wid * MS, MS)], sidx, sema)
    c2 = pltpu.async_copy(dst_hbm.at[pl.ds(wid * MS, MS)], didx, sema)
    c3 = pltpu.async_copy(ones_hbm, ones_v, sema)
    c4 = pltpu.async_copy(zeros_hbm.at[pl.ds(z0, RPT)],
                          dsrc_sh.at[pl.ds(z0, RPT)], semb)
    c5 = pltpu.async_copy(zeros_hbm.at[pl.ds(z0, RPT)],
                          ddst_sh.at[pl.ds(z0, RPT)], semb)
    c1.wait(); c2.wait(); c3.wait(); c4.wait(); c5.wait()
    plsc.subcore_barrier()

    def body(j, carry):
        @pl.when(j >= DEG_W)
        def _():
            pltpu.make_async_copy(ones_v, dsrc_sh.at[sidx.at[j - DEG_W]],
                                  sema).wait()
            pltpu.make_async_copy(ones_v, ddst_sh.at[didx.at[j - DEG_W]],
                                  semb).wait()
        pltpu.async_copy(ones_v, dsrc_sh.at[sidx.at[j]], sema, add=True)
        pltpu.async_copy(ones_v, ddst_sh.at[didx.at[j]], semb, add=True)
        return carry

    lax.fori_loop(0, MS, body, 0)

    def drain(j, carry):
        pltpu.make_async_copy(ones_v, dsrc_sh.at[sidx.at[j]], sema).wait()
        pltpu.make_async_copy(ones_v, ddst_sh.at[didx.at[j]], semb).wait()
        return carry

    lax.fori_loop(MS - DEG_W, MS, drain, 0)
    plsc.subcore_barrier()

    w1 = pltpu.async_copy(dsrc_sh.at[pl.ds(z0, RPT)],
                          out_hbm.at[c, 0, pl.ds(z0, RPT)], sema)
    w2 = pltpu.async_copy(ddst_sh.at[pl.ds(z0, RPT)],
                          out_hbm.at[c, 1, pl.ds(z0, RPT)], semb)
    w1.wait(); w2.wait()


@functools.partial(
    pl.kernel,
    out_type=jax.ShapeDtypeStruct((NC, N_PAD, D_HID), jnp.float32),
    mesh=_MESH,
    compiler_params=_SC_PARAMS,
    scratch_types=[
        pltpu.VMEM((MS, BATCH), jnp.int32),
        pltpu.VMEM((MS, BATCH), jnp.int32),
        pltpu.VMEM((RING, BATCH, D_HID), jnp.float32),
        pltpu.VMEM_SHARED((N_PAD, D_HID), jnp.float32),
        pltpu.SemaphoreType.DMA,
        pltpu.SemaphoreType.DMA,
    ],
)
def _msgpass_kernel(h_hbm, src_hbm, dst_hbm, zeros_hbm, out_hbm,
                    sidx, didx, rows, acc_sh, gsem, ssem):
    c = lax.axis_index("c")
    s = lax.axis_index("s")
    wid = c * NS + s
    z0 = s * RPT

    c1 = pltpu.async_copy(src_hbm.at[pl.ds(wid * MS, MS)], sidx, gsem)
    c2 = pltpu.async_copy(dst_hbm.at[pl.ds(wid * MS, MS)], didx, gsem)
    c3 = pltpu.async_copy(zeros_hbm.at[pl.ds(z0, RPT)],
                          acc_sh.at[pl.ds(z0, RPT)], ssem)
    c1.wait(); c2.wait(); c3.wait()
    plsc.subcore_barrier()

    _mp_loop(h_hbm, sidx, didx, rows, acc_sh, gsem, ssem)
    plsc.subcore_barrier()

    pltpu.sync_copy(acc_sh.at[pl.ds(z0, RPT)], out_hbm.at[c, pl.ds(z0, RPT)])


@functools.partial(
    pl.kernel,
    out_type=(
        jax.ShapeDtypeStruct((NC, N_PAD, D_HID), jnp.float32),
        jax.ShapeDtypeStruct((NC, N_PAD, D_HID), jnp.float32),
    ),
    mesh=_MESH,
    compiler_params=_SC_PARAMS,
    scratch_types=[
        pltpu.VMEM((MS, BATCH), jnp.int32),
        pltpu.VMEM((MS, BATCH), jnp.int32),
        pltpu.VMEM((RPT, D_HID), jnp.float32),
        pltpu.VMEM((RPT, D_HID), jnp.float32),
        pltpu.VMEM((RPT,), jnp.float32),
        pltpu.VMEM((RPT,), jnp.float32),
        pltpu.VMEM((RPT,), jnp.float32),
        pltpu.VMEM((RPT,), jnp.float32),
        pltpu.VMEM((D_HID,), jnp.float32),
        pltpu.VMEM((RING, BATCH, D_HID), jnp.float32),
        pltpu.VMEM_SHARED((N_PAD, D_HID), jnp.float32),
        pltpu.SemaphoreType.DMA,
        pltpu.SemaphoreType.DMA,
    ],
)
def _sc_b(acc1_hbm, deg_hbm, b1_hbm, src_hbm, dst_hbm, zeros2_hbm,
          h2_hbm, acc_hbm,
          sidx, didx, a0buf, a1buf, o0buf, o1buf, i0buf, i1buf, b1buf, rows,
          acc_sh, gsem, ssem):
    c = lax.axis_index("c")
    s = lax.axis_index("s")
    wid = c * NS + s
    z0 = s * RPT

    cps = [
        pltpu.async_copy(src_hbm.at[pl.ds(wid * MS, MS)], sidx, gsem),
        pltpu.async_copy(dst_hbm.at[pl.ds(wid * MS, MS)], didx, gsem),
        pltpu.async_copy(acc1_hbm.at[0, pl.ds(z0, RPT)], a0buf, gsem),
        pltpu.async_copy(acc1_hbm.at[1, pl.ds(z0, RPT)], a1buf, gsem),
        pltpu.async_copy(deg_hbm.at[0, 0, pl.ds(z0, RPT)], o0buf, gsem),
        pltpu.async_copy(deg_hbm.at[1, 0, pl.ds(z0, RPT)], o1buf, gsem),
        pltpu.async_copy(deg_hbm.at[0, 1, pl.ds(z0, RPT)], i0buf, gsem),
        pltpu.async_copy(deg_hbm.at[1, 1, pl.ds(z0, RPT)], i1buf, gsem),
        pltpu.async_copy(b1_hbm, b1buf, gsem),
        pltpu.async_copy(zeros2_hbm.at[pl.ds(z0, RPT)],
                         acc_sh.at[pl.ds(z0, RPT)], ssem),
    ]
    for cp in cps:
        cp.wait()

    b1v = b1buf[...]

    def h2_group(g, carry):
        r0 = g * D_HID
        ndv = _nrsqrt(jnp.maximum(i0buf[pl.ds(r0, D_HID)]
                                  + i1buf[pl.ds(r0, D_HID)], 1.0))
        nsv = _nrsqrt(jnp.maximum(o0buf[pl.ds(r0, D_HID)]
                                  + o1buf[pl.ds(r0, D_HID)], 1.0))
        for k in range(D_HID):
            agg = a0buf[r0 + k, :] + a1buf[r0 + k, :]
            a0buf[r0 + k, :] = jnp.maximum(agg * ndv[k] + b1v, 0.0) * nsv[k]
        return carry

    lax.fori_loop(0, RPT // D_HID, h2_group, 0)
    pltpu.sync_copy(a0buf, h2_hbm.at[c, pl.ds(z0, RPT)])
    plsc.subcore_barrier()

    _mp_loop(h2_hbm.at[c], sidx, didx, rows, acc_sh, gsem, ssem)
    plsc.subcore_barrier()

    pltpu.sync_copy(acc_sh.at[pl.ds(z0, RPT)], acc_hbm.at[c, pl.ds(z0, RPT)])


def _d1_body(feat_ref, deg_ref, w1_ref, out_ref):
    odeg = deg_ref[0, 0] + deg_ref[1, 0]
    nsrc = lax.rsqrt(jnp.maximum(odeg[:N], 1.0))
    h = feat_ref[...] * nsrc[:, None]
    out_ref[:N] = jnp.dot(h, w1_ref[...], preferred_element_type=jnp.float32)
    out_ref[N:] = jnp.zeros((N_PAD - N, D_HID), jnp.float32)


def _d3_body(accp_ref, deg_ref, w2_ref, b2_ref, out_ref):
    agg = accp_ref[0, :N] + accp_ref[1, :N]
    ideg = deg_ref[0, 1, :N] + deg_ref[1, 1, :N]
    ndst = lax.rsqrt(jnp.maximum(ideg, 1.0))[:, None]
    mm = jnp.dot(agg, w2_ref[...], preferred_element_type=jnp.float32)
    out_ref[...] = mm * ndst + b2_ref[...]


_dense1 = pl.pallas_call(
    _d1_body, out_shape=jax.ShapeDtypeStruct((N_PAD, D_HID), jnp.float32))
_dense3 = pl.pallas_call(
    _d3_body, out_shape=jax.ShapeDtypeStruct((N, D_OUT), jnp.float32))


def kernel(features, edge_index, W1, b1, W2, b2):
    src = edge_index[0].reshape(G, BATCH)
    dst = edge_index[1].reshape(G, BATCH)
    zeros1 = jnp.zeros((N_PAD,), jnp.float32)
    zeros2 = jnp.zeros((N_PAD, D_HID), jnp.float32)
    ones_b = jnp.ones((BATCH,), jnp.float32)

    degp = _deg_kernel(src, dst, zeros1, ones_b)
    h1 = _dense1(features, degp, W1)
    acc1 = _msgpass_kernel(h1, src, dst, zeros2)
    _h2, acc2 = _sc_b(acc1, degp, b1, src, dst, zeros2)
    return _dense3(acc2, degp, W2, b2)


# speedup vs baseline: 30.8836x; 1.0445x over previous
import functools

import jax
import jax.numpy as jnp
from jax import lax
from jax.experimental import pallas as pl
from jax.experimental.pallas import tpu as pltpu
from jax.experimental.pallas import tpu_sc as plsc

N = 10000
E = 320000
D_IN = 128
D_HID = 16
D_OUT = 40

NC = 2
NS = 16
NW = NC * NS
BATCH = 1000
G = E // BATCH
MS = G // NW

N_PAD = 10240
RPT = N_PAD // NS
RING = 4
DEG_W = 4

assert E == NW * MS * BATCH
assert N_PAD % (NS * 8) == 0 and N_PAD >= N

_MESH = plsc.VectorSubcoreMesh(
    core_axis_name="c", subcore_axis_name="s", num_cores=NC, num_subcores=NS
)
_SC_PARAMS = pltpu.CompilerParams(use_tc_tiling_on_sc=False)


def _nrsqrt(x):
    xi = lax.bitcast_convert_type(x, jnp.int32)
    yi = jnp.int32(0x5F3759DF) - lax.shift_right_logical(xi, 1)
    y = lax.bitcast_convert_type(yi, jnp.float32)
    for _ in range(3):
        y = y * (1.5 - 0.5 * x * y * y)
    return y


def _mp_loop(table, sidx, didx, rows, acc_sh, gsem, ssem):
    for p in range(RING - 1):
        pltpu.async_copy(table.at[sidx.at[p]], rows.at[p], gsem)

    def body(j, carry):
        pltpu.make_async_copy(table.at[sidx.at[j]], rows.at[j % RING],
                              gsem).wait()

        @pl.when(j >= 1)
        def _():
            pltpu.make_async_copy(rows.at[(j - 1) % RING],
                                  acc_sh.at[didx.at[j - 1]], ssem).wait()

        @pl.when(j + RING - 1 < MS)
        def _():
            pltpu.async_copy(table.at[sidx.at[j + RING - 1]],
                             rows.at[(j + RING - 1) % RING], gsem)

        pltpu.async_copy(rows.at[j % RING], acc_sh.at[didx.at[j]], ssem,
                         add=True)
        return carry

    lax.fori_loop(0, MS, body, 0)
    pltpu.make_async_copy(rows.at[(MS - 1) % RING],
                          acc_sh.at[didx.at[MS - 1]], ssem).wait()


@functools.partial(
    pl.kernel,
    out_type=jax.ShapeDtypeStruct((NC, 2, N_PAD), jnp.float32),
    mesh=_MESH,
    compiler_params=_SC_PARAMS,
    scratch_types=[
        pltpu.VMEM((MS, BATCH), jnp.int32),
        pltpu.VMEM((MS, BATCH), jnp.int32),
        pltpu.VMEM((BATCH,), jnp.float32),
        pltpu.VMEM_SHARED((N_PAD,), jnp.float32),
        pltpu.VMEM_SHARED((N_PAD,), jnp.float32),
        pltpu.SemaphoreType.DMA,
        pltpu.SemaphoreType.DMA,
    ],
)
def _deg_kernel(src_hbm, dst_hbm, zeros_hbm, ones_hbm, out_hbm,
                sidx, didx, ones_v, dsrc_sh, ddst_sh, sema, semb):
    c = lax.axis_index("c")
    s = lax.axis_index("s")
    wid = c * NS + s
    z0 = s * RPT

    c1 = pltpu.async_copy(src_hbm.at[pl.ds(wid * MS, MS)], sidx, sema)
    c2 = pltpu.async_copy(dst_hbm.at[pl.ds(wid * MS, MS)], didx, sema)
    c3 = pltpu.async_copy(ones_hbm, ones_v, sema)
    c4 = pltpu.async_copy(zeros_hbm.at[pl.ds(z0, RPT)],
                          dsrc_sh.at[pl.ds(z0, RPT)], semb)
    c5 = pltpu.async_copy(zeros_hbm.at[pl.ds(z0, RPT)],
                          ddst_sh.at[pl.ds(z0, RPT)], semb)
    c1.wait(); c2.wait(); c3.wait(); c4.wait(); c5.wait()
    plsc.subcore_barrier()

    def body(j, carry):
        @pl.when(j >= DEG_W)
        def _():
            pltpu.make_async_copy(ones_v, dsrc_sh.at[sidx.at[j - DEG_W]],
                                  sema).wait()
            pltpu.make_async_copy(ones_v, ddst_sh.at[didx.at[j - DEG_W]],
                                  semb).wait()
        pltpu.async_copy(ones_v, dsrc_sh.at[sidx.at[j]], sema, add=True)
        pltpu.async_copy(ones_v, ddst_sh.at[didx.at[j]], semb, add=True)
        return carry

    lax.fori_loop(0, MS, body, 0)

    def drain(j, carry):
        pltpu.make_async_copy(ones_v, dsrc_sh.at[sidx.at[j]], sema).wait()
        pltpu.make_async_copy(ones_v, ddst_sh.at[didx.at[j]], semb).wait()
        return carry

    lax.fori_loop(MS - DEG_W, MS, drain, 0)
    plsc.subcore_barrier()

    w1 = pltpu.async_copy(dsrc_sh.at[pl.ds(z0, RPT)],
                          out_hbm.at[c, 0, pl.ds(z0, RPT)], sema)
    w2 = pltpu.async_copy(ddst_sh.at[pl.ds(z0, RPT)],
                          out_hbm.at[c, 1, pl.ds(z0, RPT)], semb)
    w1.wait(); w2.wait()


@functools.partial(
    pl.kernel,
    out_type=jax.ShapeDtypeStruct((NC, N_PAD, D_HID), jnp.float32),
    mesh=_MESH,
    compiler_params=_SC_PARAMS,
    scratch_types=[
        pltpu.VMEM((MS, BATCH), jnp.int32),
        pltpu.VMEM((MS, BATCH), jnp.int32),
        pltpu.VMEM((RING, BATCH, D_HID), jnp.float32),
        pltpu.VMEM_SHARED((N_PAD, D_HID), jnp.float32),
        pltpu.SemaphoreType.DMA,
        pltpu.SemaphoreType.DMA,
    ],
)
def _msgpass_kernel(h_hbm, src_hbm, dst_hbm, zeros_hbm, out_hbm,
                    sidx, didx, rows, acc_sh, gsem, ssem):
    c = lax.axis_index("c")
    s = lax.axis_index("s")
    wid = c * NS + s
    z0 = s * RPT

    c1 = pltpu.async_copy(src_hbm.at[pl.ds(wid * MS, MS)], sidx, gsem)
    c2 = pltpu.async_copy(dst_hbm.at[pl.ds(wid * MS, MS)], didx, gsem)
    c3 = pltpu.async_copy(zeros_hbm.at[pl.ds(z0, RPT)],
                          acc_sh.at[pl.ds(z0, RPT)], ssem)
    c1.wait(); c2.wait(); c3.wait()
    plsc.subcore_barrier()

    _mp_loop(h_hbm, sidx, didx, rows, acc_sh, gsem, ssem)
    plsc.subcore_barrier()

    pltpu.sync_copy(acc_sh.at[pl.ds(z0, RPT)], out_hbm.at[c, pl.ds(z0, RPT)])


@functools.partial(
    pl.kernel,
    out_type=(
        jax.ShapeDtypeStruct((NC, N_PAD, D_HID), jnp.float32),
        jax.ShapeDtypeStruct((NC, N_PAD, D_HID), jnp.float32),
    ),
    mesh=_MESH,
    compiler_params=_SC_PARAMS,
    scratch_types=[
        pltpu.VMEM((MS, BATCH), jnp.int32),
        pltpu.VMEM((MS, BATCH), jnp.int32),
        pltpu.VMEM((RPT, D_HID), jnp.float32),
        pltpu.VMEM((RPT, D_HID), jnp.float32),
        pltpu.VMEM((RPT,), jnp.float32),
        pltpu.VMEM((RPT,), jnp.float32),
        pltpu.VMEM((RPT,), jnp.float32),
        pltpu.VMEM((RPT,), jnp.float32),
        pltpu.VMEM((D_HID,), jnp.float32),
        pltpu.VMEM((RING, BATCH, D_HID), jnp.float32),
        pltpu.VMEM_SHARED((N_PAD, D_HID), jnp.float32),
        pltpu.SemaphoreType.DMA,
        pltpu.SemaphoreType.DMA,
    ],
)
def _sc_b(acc1_hbm, deg_hbm, b1_hbm, src_hbm, dst_hbm, zeros2_hbm,
          h2_hbm, acc_hbm,
          sidx, didx, a0buf, a1buf, o0buf, o1buf, i0buf, i1buf, b1buf, rows,
          acc_sh, gsem, ssem):
    c = lax.axis_index("c")
    s = lax.axis_index("s")
    wid = c * NS + s
    z0 = s * RPT

    cps = [
        pltpu.async_copy(src_hbm.at[pl.ds(wid * MS, MS)], sidx, gsem),
        pltpu.async_copy(dst_hbm.at[pl.ds(wid * MS, MS)], didx, gsem),
        pltpu.async_copy(acc1_hbm.at[0, pl.ds(z0, RPT)], a0buf, gsem),
        pltpu.async_copy(acc1_hbm.at[1, pl.ds(z0, RPT)], a1buf, gsem),
        pltpu.async_copy(deg_hbm.at[0, 0, pl.ds(z0, RPT)], o0buf, gsem),
        pltpu.async_copy(deg_hbm.at[1, 0, pl.ds(z0, RPT)], o1buf, gsem),
        pltpu.async_copy(deg_hbm.at[0, 1, pl.ds(z0, RPT)], i0buf, gsem),
        pltpu.async_copy(deg_hbm.at[1, 1, pl.ds(z0, RPT)], i1buf, gsem),
        pltpu.async_copy(b1_hbm, b1buf, gsem),
        pltpu.async_copy(zeros2_hbm.at[pl.ds(z0, RPT)],
                         acc_sh.at[pl.ds(z0, RPT)], ssem),
    ]
    for cp in cps:
        cp.wait()

    b1v = b1buf[...]

    def h2_group(g, carry):
        r0 = g * D_HID
        ndv = _nrsqrt(jnp.maximum(i0buf[pl.ds(r0, D_HID)]
                                  + i1buf[pl.ds(r0, D_HID)], 1.0))
        nsv = _nrsqrt(jnp.maximum(o0buf[pl.ds(r0, D_HID)]
                                  + o1buf[pl.ds(r0, D_HID)], 1.0))
        for k in range(D_HID):
            agg = a0buf[r0 + k, :] + a1buf[r0 + k, :]
            a0buf[r0 + k, :] = jnp.maximum(agg * ndv[k] + b1v, 0.0) * nsv[k]
        return carry

    lax.fori_loop(0, RPT // D_HID, h2_group, 0)
    pltpu.sync_copy(a0buf, h2_hbm.at[c, pl.ds(z0, RPT)])
    plsc.subcore_barrier()

    _mp_loop(h2_hbm.at[c], sidx, didx, rows, acc_sh, gsem, ssem)
    plsc.subcore_barrier()

    pltpu.sync_copy(acc_sh.at[pl.ds(z0, RPT)], acc_hbm.at[c, pl.ds(z0, RPT)])


def _d1_body(feat_ref, deg_ref, w1_ref, out_ref):
    odeg = deg_ref[0, 0] + deg_ref[1, 0]
    nsrc = lax.rsqrt(jnp.maximum(odeg[:N], 1.0))
    h = feat_ref[...] * nsrc[:, None]
    out_ref[:N] = jnp.dot(h, w1_ref[...], preferred_element_type=jnp.float32)
    out_ref[N:] = jnp.zeros((N_PAD - N, D_HID), jnp.float32)


def _d3_body(accp_ref, deg_ref, w2_ref, b2_ref, out_ref):
    agg = accp_ref[0, :N] + accp_ref[1, :N]
    ideg = deg_ref[0, 1, :N] + deg_ref[1, 1, :N]
    ndst = lax.rsqrt(jnp.maximum(ideg, 1.0))[:, None]
    mm = jnp.dot(agg, w2_ref[...], preferred_element_type=jnp.float32)
    out_ref[...] = mm * ndst + b2_ref[...]


_dense1 = pl.pallas_call(
    _d1_body, out_shape=jax.ShapeDtypeStruct((N_PAD, D_HID), jnp.float32))
_dense3 = pl.pallas_call(
    _d3_body, out_shape=jax.ShapeDtypeStruct((N, D_OUT), jnp.float32))


def kernel(features, edge_index, W1, b1, W2, b2):
    src = edge_index[0].reshape(G, BATCH)
    dst = edge_index[1].reshape(G, BATCH)
    zeros1 = jnp.zeros((N_PAD,), jnp.float32)
    zeros2 = jnp.zeros((N_PAD, D_HID), jnp.float32)
    ones_b = jnp.ones((BATCH,), jnp.float32)

    degp = _deg_kernel(src, dst, zeros1, ones_b)
    h1 = _dense1(features, degp, W1)
    acc1 = _msgpass_kernel(h1, src, dst, zeros2)
    _h2, acc2 = _sc_b(acc1, degp, b1, src, dst, zeros2)
    return _dense3(acc2, degp, W2, b2)
